# baseline (speedup 1.0000x reference)
import functools

import jax
import jax.numpy as jnp
from jax import lax
from jax.experimental import pallas as pl
from jax.experimental.pallas import tpu as pltpu
from jax.experimental.pallas import tpu_sc as plsc

_K = 1024
_D = 256
_ROWS = 2048

_NC = 2
_NS = 16
_NW = _NC * _NS


def _tc_body(x_ref, f2_ref, e2_ref, iota_ref, emb_ref, tgt_ref, loss_ref):
    i = pl.program_id(0)
    x = x_ref[...]
    emb = emb_ref[...]
    scores2 = lax.dot_general(
        (x + x), emb, (((1,), (1,)), ((), ())),
        preferred_element_type=jnp.float32)
    t = f2_ref[...] + e2_ref[...] - scores2
    d = jnp.sqrt(jnp.maximum(t, 0.0))
    dmin = jnp.min(d, axis=1, keepdims=True)
    idxf = jnp.min(jnp.where(d == dmin, iota_ref[...], float(2 * _K)),
                   axis=1, keepdims=True)
    tgt_ref[...] = idxf.astype(jnp.int32)
    part = jnp.sum(dmin * dmin).reshape(1, 1)
    @pl.when(i == 0)
    def _():
        loss_ref[...] = part
    @pl.when(i > 0)
    def _():
        loss_ref[...] = loss_ref[...] + part


def _tc_assign(flat, f2, e2, embeddings):
    n = flat.shape[0]
    grid = (n // _ROWS,)
    call = pl.pallas_call(
        _tc_body,
        grid=grid,
        in_specs=[
            pl.BlockSpec((_ROWS, _D), lambda i: (i, 0)),
            pl.BlockSpec((_ROWS, 1), lambda i: (i, 0)),
            pl.BlockSpec((1, _K), lambda i: (0, 0)),
            pl.BlockSpec((1, _K), lambda i: (0, 0)),
            pl.BlockSpec((_K, _D), lambda i: (0, 0)),
        ],
        out_specs=[
            pl.BlockSpec((_ROWS, 1), lambda i: (i, 0)),
            pl.BlockSpec((1, 1), lambda i: (0, 0)),
        ],
        out_shape=[
            jax.ShapeDtypeStruct((n, 1), jnp.int32),
            jax.ShapeDtypeStruct((1, 1), jnp.float32),
        ],
        compiler_params=pltpu.CompilerParams(
            dimension_semantics=("arbitrary",)),
    )
    iota = lax.broadcasted_iota(jnp.float32, (1, _K), 1)
    return call(flat, f2, e2, iota, embeddings)


def _make_sc_gather(n):
    b_per_w = n // _NW
    chunk = 256
    mesh = plsc.VectorSubcoreMesh(core_axis_name="c", subcore_axis_name="s")

    nchunk = b_per_w // chunk
    nbuf = max(1, min(nchunk, (131071 - b_per_w) // (chunk * _D)))

    @functools.partial(
        pl.kernel, mesh=mesh,
        out_type=jax.ShapeDtypeStruct((n, _D), jnp.float32),
        scratch_types=(
            [pltpu.VMEM((chunk,), jnp.int32) for _ in range(nchunk)]
            + [pltpu.VMEM((chunk, _D), jnp.float32) for _ in range(nbuf)]
            + [pltpu.SemaphoreType.DMA for _ in range(2 * nbuf)]
        ),
    )
    def sc_gather(table_hbm, idx_hbm, out_hbm, *scratch):
        idx_v = scratch[:nchunk]
        rows_v = scratch[nchunk:nchunk + nbuf]
        gsem = scratch[nchunk + nbuf:nchunk + 2 * nbuf]
        ssem = scratch[nchunk + 2 * nbuf:]
        wid = lax.axis_index("s") * _NC + lax.axis_index("c")
        base = wid * b_per_w
        for c in range(nchunk):
            pltpu.sync_copy(idx_hbm.at[pl.ds(base + c * chunk, chunk)],
                            idx_v[c])
        gathers = [None] * nchunk
        stores = [None] * nchunk
        for c in range(nbuf):
            gathers[c] = pltpu.async_copy(
                table_hbm.at[idx_v[c]], rows_v[c], gsem[c])
        for c in range(nchunk):
            b = c % nbuf
            gathers[c].wait()
            stores[c] = pltpu.async_copy(
                rows_v[b], out_hbm.at[pl.ds(base + c * chunk, chunk)],
                ssem[b])
            nxt = c + nbuf
            if nxt < nchunk:
                stores[c].wait()
                gathers[nxt] = pltpu.async_copy(
                    table_hbm.at[idx_v[nxt]], rows_v[b], gsem[b])
        for c in range(max(0, nchunk - nbuf), nchunk):
            stores[c].wait()

    return sc_gather


def kernel(features, embeddings):
    B, T, D = features.shape
    flat = features.reshape(-1, D)
    n = flat.shape[0]
    f2 = jnp.sum(flat * flat, axis=1, keepdims=True)
    e2 = jnp.sum(embeddings * embeddings, axis=1)[None, :]
    h = n // 2
    gather = _make_sc_gather(h)
    tgt0, loss0 = _tc_assign(flat[:h], f2[:h], e2, embeddings)
    q0 = gather(embeddings, tgt0.reshape(h))
    tgt1, loss1 = _tc_assign(flat[h:], f2[h:], e2, embeddings)
    q1 = gather(embeddings, tgt1.reshape(h))
    quantized = jnp.concatenate([q0, q1], axis=0).reshape(B, T, D)
    targets = jnp.concatenate([tgt0, tgt1], axis=0).reshape(B, T)
    extra_losses = (0.25 / (n * D)) * (loss0[0, 0] + loss1[0, 0])
    return quantized, targets, extra_losses


# speedup vs baseline: 1.4221x; 1.4221x over previous
import functools

import jax
import jax.numpy as jnp
from jax import lax
from jax.experimental import pallas as pl
from jax.experimental.pallas import tpu as pltpu
from jax.experimental.pallas import tpu_sc as plsc

_K = 1024
_D = 256
_ROWS = 4096

_NC = 2
_NS = 16
_NW = _NC * _NS


def _tc_body(x_ref, f2_ref, e2_ref, iota_ref, emb_ref, tgt_ref, loss_ref):
    i = pl.program_id(0)
    x = x_ref[...]
    emb = emb_ref[...]
    scores2 = lax.dot_general(
        (x + x), emb, (((1,), (1,)), ((), ())),
        preferred_element_type=jnp.float32)
    t = f2_ref[...] + e2_ref[...] - scores2
    d = jnp.sqrt(jnp.maximum(t, 0.0))
    dmin = jnp.min(d, axis=1, keepdims=True)
    idxf = jnp.min(jnp.where(d == dmin, iota_ref[...], float(2 * _K)),
                   axis=1, keepdims=True)
    tgt_ref[...] = idxf.astype(jnp.int32)
    part = jnp.sum(dmin * dmin).reshape(1, 1)
    @pl.when(i == 0)
    def _():
        loss_ref[...] = part
    @pl.when(i > 0)
    def _():
        loss_ref[...] = loss_ref[...] + part


def _tc_assign(flat, f2, e2, embeddings):
    n = flat.shape[0]
    grid = (n // _ROWS,)
    call = pl.pallas_call(
        _tc_body,
        grid=grid,
        in_specs=[
            pl.BlockSpec((_ROWS, _D), lambda i: (i, 0)),
            pl.BlockSpec((_ROWS, 1), lambda i: (i, 0)),
            pl.BlockSpec((1, _K), lambda i: (0, 0)),
            pl.BlockSpec((1, _K), lambda i: (0, 0)),
            pl.BlockSpec((_K, _D), lambda i: (0, 0)),
        ],
        out_specs=[
            pl.BlockSpec((_ROWS, 1), lambda i: (i, 0)),
            pl.BlockSpec((1, 1), lambda i: (0, 0)),
        ],
        out_shape=[
            jax.ShapeDtypeStruct((n, 1), jnp.int32),
            jax.ShapeDtypeStruct((1, 1), jnp.float32),
        ],
        compiler_params=pltpu.CompilerParams(
            dimension_semantics=("arbitrary",)),
    )
    iota = lax.broadcasted_iota(jnp.float32, (1, _K), 1)
    return call(flat, f2, e2, iota, embeddings)


def _make_sc_gather(n):
    b_per_w = n // _NW
    chunk = 256
    mesh = plsc.VectorSubcoreMesh(core_axis_name="c", subcore_axis_name="s")

    nchunk = b_per_w // chunk
    nbuf = max(1, min(nchunk, (131071 - b_per_w) // (chunk * _D)))

    @functools.partial(
        pl.kernel, mesh=mesh,
        out_type=jax.ShapeDtypeStruct((n, _D), jnp.float32),
        scratch_types=(
            [pltpu.VMEM((chunk,), jnp.int32) for _ in range(nchunk)]
            + [pltpu.VMEM((chunk, _D), jnp.float32) for _ in range(nbuf)]
            + [pltpu.SemaphoreType.DMA for _ in range(2 * nbuf)]
        ),
    )
    def sc_gather(table_hbm, idx_hbm, out_hbm, *scratch):
        idx_v = scratch[:nchunk]
        rows_v = scratch[nchunk:nchunk + nbuf]
        gsem = scratch[nchunk + nbuf:nchunk + 2 * nbuf]
        ssem = scratch[nchunk + 2 * nbuf:]
        wid = lax.axis_index("s") * _NC + lax.axis_index("c")
        base = wid * b_per_w
        for c in range(nchunk):
            pltpu.sync_copy(idx_hbm.at[pl.ds(base + c * chunk, chunk)],
                            idx_v[c])
        gathers = [None] * nchunk
        stores = [None] * nchunk
        for c in range(nbuf):
            gathers[c] = pltpu.async_copy(
                table_hbm.at[idx_v[c]], rows_v[c], gsem[c])
        for c in range(nchunk):
            b = c % nbuf
            gathers[c].wait()
            stores[c] = pltpu.async_copy(
                rows_v[b], out_hbm.at[pl.ds(base + c * chunk, chunk)],
                ssem[b])
            nxt = c + nbuf
            if nxt < nchunk:
                stores[c].wait()
                gathers[nxt] = pltpu.async_copy(
                    table_hbm.at[idx_v[nxt]], rows_v[b], gsem[b])
        for c in range(max(0, nchunk - nbuf), nchunk):
            stores[c].wait()

    return sc_gather


def kernel(features, embeddings):
    B, T, D = features.shape
    flat = features.reshape(-1, D)
    n = flat.shape[0]
    f2 = jnp.sum(flat * flat, axis=1, keepdims=True)
    e2 = jnp.sum(embeddings * embeddings, axis=1)[None, :]
    tgt, loss_sum = _tc_assign(flat, f2, e2, embeddings)
    quantized_flat = _make_sc_gather(n)(embeddings, tgt.reshape(n))
    quantized = quantized_flat.reshape(B, T, D)
    targets = tgt.reshape(B, T)
    extra_losses = (0.25 / (n * D)) * loss_sum[0, 0]
    return quantized, targets, extra_losses
